# baseline (speedup 1.0000x reference)
import jax
import jax.numpy as jnp
from jax import lax
from jax.experimental import pallas as pl
from jax.experimental.pallas import tpu as pltpu
from jax.experimental.pallas import tpu_sc as plsc

NIMG = 8
PIX = 512 * 512
NWORK = 32
PER_W = NIMG * PIX // NWORK
CHUNK = 8192
NBINS = 4096
NLANE = 16
OFF = (115 << 8) - 1
OUT_W = 2 * NBINS
UNROLL = 4


ROWS_PER_CHUNK = CHUNK // 512


def _sc_hist_body(lg_hbm, lb_hbm, out_hbm, lbuf, abuf, hist, obuf, sems):
    wid = lax.axis_index("c") * 16 + lax.axis_index("s")
    region = lax.iota(jnp.int32, NLANE) * NBINS
    img = wid // 4
    row0 = (wid % 4) * 128
    nch = PER_W // CHUNK

    def issue(c, b):
        rs = row0 + c * ROWS_PER_CHUNK
        dst = pl.ds(b * ROWS_PER_CHUNK, ROWS_PER_CHUNK)
        return (
            pltpu.async_copy(lg_hbm.at[img, pl.ds(rs, ROWS_PER_CHUNK), :],
                             lbuf.at[dst, :], sems.at[b]),
            pltpu.async_copy(lb_hbm.at[img, pl.ds(rs, ROWS_PER_CHUNK), :],
                             abuf.at[dst, :], sems.at[b]),
        )

    pending = issue(0, 0)

    @plsc.parallel_loop(0, NBINS * NLANE, step=NLANE, unroll=8)
    def _zero(j):
        hist[pl.ds(j, NLANE)] = jnp.zeros((NLANE,), jnp.int32)

    for c in range(nch):
        b = c & 1
        for h in pending:
            h.wait()
        if c + 1 < nch:
            pending = issue(c + 1, 1 - b)

        @plsc.parallel_loop(0, 512, step=NLANE, unroll=1)
        def _body(j):
            sl = pl.ds(j, NLANE)
            for r in range(ROWS_PER_CHUNK):
                row = b * ROWS_PER_CHUNK + r
                lg = lbuf[row, sl]
                lb = abuf[row, sl]
                e = 1.0 - lg * (2.0 * lb.astype(jnp.float32) - 1.0)
                bits = lax.bitcast_convert_type(e, jnp.int32)
                bn = jnp.minimum(lax.shift_right_logical(bits, 15) - OFF,
                                 NBINS - 1)
                bn = jnp.where(e < jnp.float32(2.0 ** -12),
                               jnp.zeros((NLANE,), jnp.int32), bn)
                plsc.addupdate_scatter(hist, [region + bn], 1 + lb * 65536)

    @plsc.parallel_loop(0, NBINS, step=NLANE, unroll=2)
    def _reduce(j):
        v = hist[pl.ds(j, NLANE)]
        acc_c = jnp.bitwise_and(v, 65535)
        acc_p = lax.shift_right_logical(v, 16)
        for r in range(1, NLANE):
            v = hist[pl.ds(r * NBINS + j, NLANE)]
            acc_c = acc_c + jnp.bitwise_and(v, 65535)
            acc_p = acc_p + lax.shift_right_logical(v, 16)
        obuf[pl.ds(j, NLANE)] = acc_c
        obuf[pl.ds(NBINS + j, NLANE)] = acc_p

    pltpu.sync_copy(obuf, out_hbm.at[pl.ds(wid * OUT_W, OUT_W)])


def _tc_finish_body(hist_ref, out_ref):
    dot = lambda a, b: jnp.dot(a, b, precision=jax.lax.Precision.HIGHEST,
                               preferred_element_type=jnp.float32)
    i128 = lax.broadcasted_iota(jnp.int32, (128, 128), 0)
    j128 = lax.broadcasted_iota(jnp.int32, (128, 128), 1)
    l128 = (i128 <= j128).astype(jnp.float32)
    i32_ = lax.broadcasted_iota(jnp.int32, (32, 32), 0)
    j32_ = lax.broadcasted_iota(jnp.int32, (32, 32), 1)
    e32 = (j32_ < i32_).astype(jnp.float32)
    ones128 = jnp.ones((128, 1), jnp.float32)

    g = (lax.broadcasted_iota(jnp.int32, (32, 128), 0) * 128
         + lax.broadcasted_iota(jnp.int32, (32, 128), 1))

    def center(gg):
        u = lax.shift_left(gg + OFF, 15) | (1 << 14)
        return jnp.where(gg < 1, 0.0, lax.bitcast_convert_type(u, jnp.float32))

    da = center(g) - center(g - 1)

    def cum(z):
        within = dot(z, l128)
        totals = dot(z, ones128)
        return within + dot(e32, totals), totals

    total = jnp.float32(0.0)
    for i in range(NIMG):
        xw = (hist_ref[4 * i] + hist_ref[4 * i + 1]
              + hist_ref[4 * i + 2] + hist_ref[4 * i + 3])
        cnt = xw[0:32].astype(jnp.float32)
        pos = xw[32:64].astype(jnp.float32)
        casc, tc_ = cum(cnt)
        pasc, tp_ = cum(pos)
        tsum = jnp.sum(tc_)
        gsum = jnp.sum(tp_)
        n_b = tsum - casc + cnt
        cg_b = gsum - pasc + pos
        denom = jnp.maximum(gsum + n_b - cg_b, 1.0)
        jac = jnp.where(n_b > 0, 1.0 - (gsum - cg_b) / denom, 0.0)
        total = total + jnp.sum(jac * da)

    out_ref[...] = jnp.full((8, 128), total / NIMG, jnp.float32)


def kernel(logits, labels):
    lg = logits.astype(jnp.float32)
    lb = labels

    mesh = plsc.VectorSubcoreMesh(core_axis_name="c", subcore_axis_name="s")
    hist = pl.kernel(
        _sc_hist_body,
        mesh=mesh,
        compiler_params=pltpu.CompilerParams(needs_layout_passes=False,
                                             disable_bounds_checks=True),
        out_type=jax.ShapeDtypeStruct((NWORK * OUT_W,), jnp.int32),
        scratch_types=[
            pltpu.VMEM((2 * ROWS_PER_CHUNK, 512), jnp.float32),
            pltpu.VMEM((2 * ROWS_PER_CHUNK, 512), jnp.int32),
            pltpu.VMEM((NBINS * NLANE,), jnp.int32),
            pltpu.VMEM((OUT_W,), jnp.int32),
            pltpu.SemaphoreType.DMA((2,)),
        ],
    )(lg, lb)

    hist3 = hist.reshape(NWORK, OUT_W // 128, 128)
    out = pl.pallas_call(
        _tc_finish_body,
        out_shape=jax.ShapeDtypeStruct((8, 128), jnp.float32),
    )(hist3)

    return out[0, 0]


# speedup vs baseline: 1.0841x; 1.0841x over previous
import jax
import jax.numpy as jnp
from jax import lax
from jax.experimental import pallas as pl
from jax.experimental.pallas import tpu as pltpu
from jax.experimental.pallas import tpu_sc as plsc

NIMG = 8
PIX = 512 * 512
NWORK = 32
PER_W = NIMG * PIX // NWORK
CHUNK = 8192
NBINS = 4096
NLANE = 16
OFF = (115 << 8) - 1
OUT_W = 2 * NBINS
UNROLL = 4


ROWS_PER_CHUNK = CHUNK // 512


def _sc_hist_body(lg_hbm, lb_hbm, out_hbm, lbuf, abuf, hist, obuf, sems):
    wid = lax.axis_index("c") * 16 + lax.axis_index("s")
    region = lax.iota(jnp.int32, NLANE) * NBINS
    img = wid // 4
    row0 = (wid % 4) * 128
    nch = PER_W // CHUNK

    def issue(c, b):
        rs = row0 + c * ROWS_PER_CHUNK
        dst = pl.ds(b * ROWS_PER_CHUNK, ROWS_PER_CHUNK)
        return (
            pltpu.async_copy(lg_hbm.at[img, pl.ds(rs, ROWS_PER_CHUNK), :],
                             lbuf.at[dst, :], sems.at[b]),
            pltpu.async_copy(lb_hbm.at[img, pl.ds(rs, ROWS_PER_CHUNK), :],
                             abuf.at[dst, :], sems.at[b]),
        )

    pending = issue(0, 0)

    @plsc.parallel_loop(0, NBINS * NLANE, step=NLANE, unroll=8)
    def _zero(j):
        hist[pl.ds(j, NLANE)] = jnp.zeros((NLANE,), jnp.int32)

    for c in range(nch):
        b = c & 1
        for h in pending:
            h.wait()
        if c + 1 < nch:
            pending = issue(c + 1, 1 - b)

        @plsc.parallel_loop(0, CHUNK, step=NLANE, unroll=UNROLL)
        def _body(i):
            k = i + lax.iota(jnp.int32, NLANE)
            rows = b * ROWS_PER_CHUNK + lax.shift_right_logical(k, 9)
            cols = jnp.bitwise_and(k, 511)
            lg = plsc.load_gather(lbuf, [rows, cols])
            lb = plsc.load_gather(abuf, [rows, cols])
            e = 1.0 - lg * (2.0 * lb.astype(jnp.float32) - 1.0)
            bits = lax.bitcast_convert_type(e, jnp.int32)
            bn = jnp.clip(lax.shift_right_arithmetic(bits, 15) - OFF,
                          0, NBINS - 1)
            plsc.addupdate_scatter(hist, [region + bn], 1 + lb * 65536)

    @plsc.parallel_loop(0, NBINS, step=NLANE, unroll=2)
    def _reduce(j):
        v = hist[pl.ds(j, NLANE)]
        acc_c = jnp.bitwise_and(v, 65535)
        acc_p = lax.shift_right_logical(v, 16)
        for r in range(1, NLANE):
            v = hist[pl.ds(r * NBINS + j, NLANE)]
            acc_c = acc_c + jnp.bitwise_and(v, 65535)
            acc_p = acc_p + lax.shift_right_logical(v, 16)
        obuf[pl.ds(j, NLANE)] = acc_c
        obuf[pl.ds(NBINS + j, NLANE)] = acc_p

    pltpu.sync_copy(obuf, out_hbm.at[pl.ds(wid * OUT_W, OUT_W)])


def _tc_finish_body(hist_ref, out_ref):
    dot = lambda a, b: jnp.dot(a, b, precision=jax.lax.Precision.HIGHEST,
                               preferred_element_type=jnp.float32)
    i128 = lax.broadcasted_iota(jnp.int32, (128, 128), 0)
    j128 = lax.broadcasted_iota(jnp.int32, (128, 128), 1)
    l128 = (i128 <= j128).astype(jnp.float32)
    i32_ = lax.broadcasted_iota(jnp.int32, (32, 32), 0)
    j32_ = lax.broadcasted_iota(jnp.int32, (32, 32), 1)
    e32 = (j32_ < i32_).astype(jnp.float32)
    ones128 = jnp.ones((128, 1), jnp.float32)

    g = (lax.broadcasted_iota(jnp.int32, (32, 128), 0) * 128
         + lax.broadcasted_iota(jnp.int32, (32, 128), 1))

    def center(gg):
        u = lax.shift_left(gg + OFF, 15) | (1 << 14)
        return jnp.where(gg < 1, 0.0, lax.bitcast_convert_type(u, jnp.float32))

    da = center(g) - center(g - 1)

    def cum(z):
        within = dot(z, l128)
        totals = dot(z, ones128)
        return within + dot(e32, totals), totals

    total = jnp.float32(0.0)
    for i in range(NIMG):
        xw = (hist_ref[4 * i] + hist_ref[4 * i + 1]
              + hist_ref[4 * i + 2] + hist_ref[4 * i + 3])
        cnt = xw[0:32].astype(jnp.float32)
        pos = xw[32:64].astype(jnp.float32)
        casc, tc_ = cum(cnt)
        pasc, tp_ = cum(pos)
        tsum = jnp.sum(tc_)
        gsum = jnp.sum(tp_)
        n_b = tsum - casc + cnt
        cg_b = gsum - pasc + pos
        denom = jnp.maximum(gsum + n_b - cg_b, 1.0)
        jac = jnp.where(n_b > 0, 1.0 - (gsum - cg_b) / denom, 0.0)
        total = total + jnp.sum(jac * da)

    out_ref[...] = jnp.full((8, 128), total / NIMG, jnp.float32)


def kernel(logits, labels):
    lg = logits.astype(jnp.float32)
    lb = labels

    mesh = plsc.VectorSubcoreMesh(core_axis_name="c", subcore_axis_name="s")
    hist = pl.kernel(
        _sc_hist_body,
        mesh=mesh,
        compiler_params=pltpu.CompilerParams(needs_layout_passes=False,
                                             disable_bounds_checks=True),
        out_type=jax.ShapeDtypeStruct((NWORK * OUT_W,), jnp.int32),
        scratch_types=[
            pltpu.VMEM((2 * ROWS_PER_CHUNK, 512), jnp.float32),
            pltpu.VMEM((2 * ROWS_PER_CHUNK, 512), jnp.int32),
            pltpu.VMEM((NBINS * NLANE,), jnp.int32),
            pltpu.VMEM((OUT_W,), jnp.int32),
            pltpu.SemaphoreType.DMA((2,)),
        ],
    )(lg, lb)

    hist3 = hist.reshape(NWORK, OUT_W // 128, 128)
    out = pl.pallas_call(
        _tc_finish_body,
        out_shape=jax.ShapeDtypeStruct((8, 128), jnp.float32),
    )(hist3)

    return out[0, 0]


# speedup vs baseline: 1.2352x; 1.1395x over previous
import jax
import jax.numpy as jnp
from jax import lax
from jax.experimental import pallas as pl
from jax.experimental.pallas import tpu as pltpu
from jax.experimental.pallas import tpu_sc as plsc

NIMG = 8
PIX = 512 * 512
NWORK = 32
PER_W = NIMG * PIX // NWORK
CHUNK = 8192
NBINS = 4096
NLANE = 16
OFF = (115 << 8) - 1
OUT_W = 2 * NBINS
UNROLL = 8


ROWS_PER_CHUNK = CHUNK // 512


def _sc_hist_body(lg_hbm, lb_hbm, out_hbm, lbuf, abuf, hist, obuf, sems):
    wid = lax.axis_index("c") * 16 + lax.axis_index("s")
    region = lax.iota(jnp.int32, NLANE) * NBINS
    img = wid // 4
    row0 = (wid % 4) * 128
    nch = PER_W // CHUNK

    def issue(c, b):
        rs = row0 + c * ROWS_PER_CHUNK
        dst = pl.ds(b * ROWS_PER_CHUNK, ROWS_PER_CHUNK)
        return (
            pltpu.async_copy(lg_hbm.at[img, pl.ds(rs, ROWS_PER_CHUNK), :],
                             lbuf.at[dst, :], sems.at[b]),
            pltpu.async_copy(lb_hbm.at[img, pl.ds(rs, ROWS_PER_CHUNK), :],
                             abuf.at[dst, :], sems.at[b]),
        )

    pending = issue(0, 0)

    @plsc.parallel_loop(0, NBINS * NLANE, step=NLANE, unroll=8)
    def _zero(j):
        hist[pl.ds(j, NLANE)] = jnp.zeros((NLANE,), jnp.int32)

    for c in range(nch):
        b = c & 1
        for h in pending:
            h.wait()
        if c + 1 < nch:
            pending = issue(c + 1, 1 - b)

        @plsc.parallel_loop(0, CHUNK, step=NLANE, unroll=UNROLL)
        def _body(i):
            k = i + lax.iota(jnp.int32, NLANE)
            rows = b * ROWS_PER_CHUNK + lax.shift_right_logical(k, 9)
            cols = jnp.bitwise_and(k, 511)
            lg = plsc.load_gather(lbuf, [rows, cols])
            lb = plsc.load_gather(abuf, [rows, cols])
            e = 1.0 - lg * (2.0 * lb.astype(jnp.float32) - 1.0)
            bits = lax.bitcast_convert_type(e, jnp.int32)
            bn = jnp.clip(lax.shift_right_arithmetic(bits, 15) - OFF,
                          0, NBINS - 1)
            plsc.addupdate_scatter(hist, [region + bn], 1 + lb * 65536)

    @plsc.parallel_loop(0, NBINS, step=NLANE, unroll=2)
    def _reduce(j):
        v = hist[pl.ds(j, NLANE)]
        acc_c = jnp.bitwise_and(v, 65535)
        acc_p = lax.shift_right_logical(v, 16)
        for r in range(1, NLANE):
            v = hist[pl.ds(r * NBINS + j, NLANE)]
            acc_c = acc_c + jnp.bitwise_and(v, 65535)
            acc_p = acc_p + lax.shift_right_logical(v, 16)
        obuf[pl.ds(j, NLANE)] = acc_c
        obuf[pl.ds(NBINS + j, NLANE)] = acc_p

    pltpu.sync_copy(obuf, out_hbm.at[pl.ds(wid * OUT_W, OUT_W)])


def _tc_finish_body(hist_ref, out_ref):
    dot = lambda a, b: jnp.dot(a, b, precision=jax.lax.Precision.HIGHEST,
                               preferred_element_type=jnp.float32)
    i128 = lax.broadcasted_iota(jnp.int32, (128, 128), 0)
    j128 = lax.broadcasted_iota(jnp.int32, (128, 128), 1)
    l128 = (i128 <= j128).astype(jnp.float32)
    i32_ = lax.broadcasted_iota(jnp.int32, (32, 32), 0)
    j32_ = lax.broadcasted_iota(jnp.int32, (32, 32), 1)
    e32 = (j32_ < i32_).astype(jnp.float32)
    ones128 = jnp.ones((128, 1), jnp.float32)

    g = (lax.broadcasted_iota(jnp.int32, (32, 128), 0) * 128
         + lax.broadcasted_iota(jnp.int32, (32, 128), 1))

    def center(gg):
        u = lax.shift_left(gg + OFF, 15) | (1 << 14)
        return jnp.where(gg < 1, 0.0, lax.bitcast_convert_type(u, jnp.float32))

    da = center(g) - center(g - 1)

    def cum(z):
        within = dot(z, l128)
        totals = dot(z, ones128)
        return within + dot(e32, totals), totals

    total = jnp.float32(0.0)
    for i in range(NIMG):
        xw = (hist_ref[4 * i] + hist_ref[4 * i + 1]
              + hist_ref[4 * i + 2] + hist_ref[4 * i + 3])
        cnt = xw[0:32].astype(jnp.float32)
        pos = xw[32:64].astype(jnp.float32)
        casc, tc_ = cum(cnt)
        pasc, tp_ = cum(pos)
        tsum = jnp.sum(tc_)
        gsum = jnp.sum(tp_)
        n_b = tsum - casc + cnt
        cg_b = gsum - pasc + pos
        denom = jnp.maximum(gsum + n_b - cg_b, 1.0)
        jac = jnp.where(n_b > 0, 1.0 - (gsum - cg_b) / denom, 0.0)
        total = total + jnp.sum(jac * da)

    out_ref[...] = jnp.full((8, 128), total / NIMG, jnp.float32)


def kernel(logits, labels):
    lg = logits.astype(jnp.float32)
    lb = labels

    mesh = plsc.VectorSubcoreMesh(core_axis_name="c", subcore_axis_name="s")
    hist = pl.kernel(
        _sc_hist_body,
        mesh=mesh,
        compiler_params=pltpu.CompilerParams(needs_layout_passes=False,
                                             disable_bounds_checks=True),
        out_type=jax.ShapeDtypeStruct((NWORK * OUT_W,), jnp.int32),
        scratch_types=[
            pltpu.VMEM((2 * ROWS_PER_CHUNK, 512), jnp.float32),
            pltpu.VMEM((2 * ROWS_PER_CHUNK, 512), jnp.int32),
            pltpu.VMEM((NBINS * NLANE,), jnp.int32),
            pltpu.VMEM((OUT_W,), jnp.int32),
            pltpu.SemaphoreType.DMA((2,)),
        ],
    )(lg, lb)

    hist3 = hist.reshape(NWORK, OUT_W // 128, 128)
    out = pl.pallas_call(
        _tc_finish_body,
        out_shape=jax.ShapeDtypeStruct((8, 128), jnp.float32),
    )(hist3)

    return out[0, 0]
